# Initial kernel scaffold; baseline (speedup 1.0000x reference)
import jax
import jax.numpy as jnp
from jax.experimental import pallas as pl


def kernel(x, edge_index, edge_attr, batch, params):
    raise NotImplementedError("write your pallas kernel here")


# speedup vs baseline: 1.7758x; 1.7758x over previous
import functools

import jax
import jax.numpy as jnp
from jax import lax
from jax.experimental import pallas as pl
from jax.experimental.pallas import tpu as pltpu
from jax.experimental.pallas import tpu_sc as plsc

_N = 10000
_E = 320000
_D = 128
_ED = 16
_H = 128
_G = 64

_NC = 2
_NS = 16
_NW = _NC * _NS
_CHUNK = 128
_EPAD = 327680
_PERW = _EPAD // _NW
_NCHUNK = _PERW // _CHUNK
_NPAD = 10240
_RPT = _NPAD // _NS

_BN = 512
_BE = 1024
_F32 = jnp.float32
_PREC = lax.Precision.DEFAULT


def _mesh():
    return plsc.VectorSubcoreMesh(core_axis_name="c", subcore_axis_name="s")


def _sc_gather(table, idx):
    @functools.partial(
        pl.kernel,
        out_type=jax.ShapeDtypeStruct((_EPAD, _H), _F32),
        mesh=_mesh(),
        scratch_types=[
            pltpu.VMEM((_NCHUNK, _CHUNK), jnp.int32),
            pltpu.VMEM((_CHUNK, _H), _F32),
            pltpu.VMEM((_CHUNK, _H), _F32),
            pltpu.SemaphoreType.DMA,
            pltpu.SemaphoreType.DMA,
        ],
    )
    def k(table_hbm, idx_hbm, out_hbm, idx_v, buf0, buf1, sem0, sem1):
        c = lax.axis_index("c")
        s = lax.axis_index("s")
        wid = s * _NC + c
        base = wid * _PERW
        pltpu.sync_copy(idx_hbm.at[wid], idx_v)
        bufs = (buf0, buf1)
        sems = (sem0, sem1)

        def start(j, b):
            pltpu.async_copy(table_hbm.at[idx_v.at[j]], bufs[b], sems[b])

        start(0, 0)
        start(1, 1)

        def body(jj, carry):
            for b in range(2):
                j = jj * 2 + b
                pltpu.make_async_copy(
                    table_hbm.at[idx_v.at[j]], bufs[b], sems[b]).wait()
                pltpu.sync_copy(
                    bufs[b], out_hbm.at[pl.ds(base + j * _CHUNK, _CHUNK)])

                @pl.when(j + 2 < _NCHUNK)
                def _():
                    start(j + 2, b)
            return carry

        lax.fori_loop(0, _NCHUNK // 2, body, 0)

    return k(table, idx)


def _sc_scatter_add(m, idx, zblk):
    @functools.partial(
        pl.kernel,
        out_type=jax.ShapeDtypeStruct((_NC, _NPAD, _H), _F32),
        mesh=_mesh(),
        scratch_types=[
            pltpu.VMEM((_NCHUNK, _CHUNK), jnp.int32),
            pltpu.VMEM((_CHUNK, _H), _F32),
            pltpu.VMEM((_CHUNK, _H), _F32),
            pltpu.VMEM_SHARED((_NPAD, _H), _F32),
            pltpu.SemaphoreType.DMA,
            pltpu.SemaphoreType.DMA,
        ],
    )
    def k(m_hbm, idx_hbm, z_hbm, out_hbm, idx_v, buf0, buf1, acc, sem0, sem1):
        c = lax.axis_index("c")
        s = lax.axis_index("s")
        wid = s * _NC + c
        base = wid * _PERW
        for r in range(_RPT // _CHUNK):
            pltpu.sync_copy(
                z_hbm, acc.at[pl.ds(s * _RPT + r * _CHUNK, _CHUNK)])
        pltpu.sync_copy(idx_hbm.at[wid], idx_v)
        plsc.subcore_barrier()

        bufs = (buf0, buf1)
        sems = (sem0, sem1)

        def start(j, b):
            pltpu.async_copy(
                m_hbm.at[pl.ds(base + j * _CHUNK, _CHUNK)], bufs[b], sems[b])

        start(0, 0)
        start(1, 1)

        def body(jj, carry):
            for b in range(2):
                j = jj * 2 + b
                pltpu.make_async_copy(
                    m_hbm.at[pl.ds(base + j * _CHUNK, _CHUNK)],
                    bufs[b], sems[b]).wait()
                pltpu.sync_copy(bufs[b], acc.at[idx_v.at[j]], add=True)

                @pl.when(j + 2 < _NCHUNK)
                def _():
                    start(j + 2, b)
            return carry

        lax.fori_loop(0, _NCHUNK // 2, body, 0)
        plsc.subcore_barrier()
        pltpu.sync_copy(
            acc.at[pl.ds(s * _RPT, _RPT)],
            out_hbm.at[c, pl.ds(s * _RPT, _RPT)])

    return k(m, idx, zblk)


def _tc_params():
    return pltpu.CompilerParams(dimension_semantics=("arbitrary",))


def _tc_dense(x, w, b):
    n, din = x.shape
    dout = w.shape[1]

    def body(x_ref, w_ref, b_ref, o_ref):
        o_ref[...] = jnp.dot(
            x_ref[...], w_ref[...], preferred_element_type=_F32, precision=_PREC) + b_ref[...]

    return pl.pallas_call(
        body,
        grid=(pl.cdiv(n, _BN),),
        in_specs=[
            pl.BlockSpec((_BN, din), lambda i: (i, 0)),
            pl.BlockSpec((din, dout), lambda i: (0, 0)),
            pl.BlockSpec((1, dout), lambda i: (0, 0)),
        ],
        out_specs=pl.BlockSpec((_BN, dout), lambda i: (i, 0)),
        out_shape=jax.ShapeDtypeStruct((n, dout), _F32),
        compiler_params=_tc_params(),
    )(x, w, b.reshape(1, -1))


def _tc_edge_mlp(xj, ea, w1x, w1e, b1, w2, b2):
    def body(xj_ref, ea_ref, w1x_ref, w1e_ref, b1_ref, w2_ref, b2_ref, o_ref):
        t = jnp.dot(xj_ref[...], w1x_ref[...], preferred_element_type=_F32, precision=_PREC)
        t += jnp.dot(ea_ref[...], w1e_ref[...], preferred_element_type=_F32, precision=_PREC)
        t = jnp.maximum(t + b1_ref[...], 0.0)
        o_ref[...] = jnp.dot(
            t, w2_ref[...], preferred_element_type=_F32, precision=_PREC) + b2_ref[...]

    return pl.pallas_call(
        body,
        grid=(_EPAD // _BE,),
        in_specs=[
            pl.BlockSpec((_BE, _H), lambda i: (i, 0)),
            pl.BlockSpec((_BE, _ED), lambda i: (i, 0)),
            pl.BlockSpec((_H, _H), lambda i: (0, 0)),
            pl.BlockSpec((_ED, _H), lambda i: (0, 0)),
            pl.BlockSpec((1, _H), lambda i: (0, 0)),
            pl.BlockSpec((_H, _H), lambda i: (0, 0)),
            pl.BlockSpec((1, _H), lambda i: (0, 0)),
        ],
        out_specs=pl.BlockSpec((_BE, _H), lambda i: (i, 0)),
        out_shape=jax.ShapeDtypeStruct((_EPAD, _H), _F32),
        compiler_params=_tc_params(),
    )(xj, ea, w1x, w1e, b1.reshape(1, -1), w2, b2.reshape(1, -1))


def _tc_update(h, a0, a1, uwh, uwa, ub, lng, lnb):
    def body(h_ref, a0_ref, a1_ref, uwh_ref, uwa_ref, ub_ref, g_ref, b_ref,
             o_ref):
        hb = h_ref[...]
        ag = a0_ref[...] + a1_ref[...]
        u = jnp.dot(hb, uwh_ref[...], preferred_element_type=_F32, precision=_PREC)
        u += jnp.dot(ag, uwa_ref[...], preferred_element_type=_F32, precision=_PREC)
        u = jnp.maximum(u + ub_ref[...], 0.0)
        mu = jnp.mean(u, axis=1, keepdims=True)
        var = jnp.mean((u - mu) ** 2, axis=1, keepdims=True)
        hn = (u - mu) / jnp.sqrt(var + 1e-5) * g_ref[...] + b_ref[...]
        o_ref[...] = jnp.maximum(hn, 0.0) + hb

    return pl.pallas_call(
        body,
        grid=(pl.cdiv(_N, _BN),),
        in_specs=[
            pl.BlockSpec((_BN, _H), lambda i: (i, 0)),
            pl.BlockSpec((_BN, _H), lambda i: (i, 0)),
            pl.BlockSpec((_BN, _H), lambda i: (i, 0)),
            pl.BlockSpec((_H, _H), lambda i: (0, 0)),
            pl.BlockSpec((_H, _H), lambda i: (0, 0)),
            pl.BlockSpec((1, _H), lambda i: (0, 0)),
            pl.BlockSpec((1, _H), lambda i: (0, 0)),
            pl.BlockSpec((1, _H), lambda i: (0, 0)),
        ],
        out_specs=pl.BlockSpec((_BN, _H), lambda i: (i, 0)),
        out_shape=jax.ShapeDtypeStruct((_N, _H), _F32),
        compiler_params=_tc_params(),
    )(h, a0, a1, uwh, uwa, ub.reshape(1, -1), lng.reshape(1, -1),
      lnb.reshape(1, -1))


def _tc_pool_readout(h_pad, brow, bcol, f1m, f1x, f1b, f2w, f2b):
    grid = _NPAD // _BN

    def body(h_ref, br_ref, bc_ref, f1m_ref, f1x_ref, f1b_ref, f2w_ref,
             f2b_ref, o_ref, sum_s, max_s, cnt_s):
        i = pl.program_id(0)

        @pl.when(i == 0)
        def _():
            sum_s[...] = jnp.zeros_like(sum_s)
            max_s[...] = jnp.full_like(max_s, -jnp.inf)
            cnt_s[...] = jnp.zeros_like(cnt_s)

        hb = h_ref[...]
        bt = br_ref[0]
        btc = bc_ref[0]
        gi = lax.broadcasted_iota(jnp.int32, (_G, 1), 0)
        onehot = (gi == bt).astype(_F32)
        sum_s[...] += jnp.dot(onehot, hb, preferred_element_type=_F32,
                              precision=lax.Precision.HIGHEST)
        cnt_s[...] += jnp.sum(onehot, axis=1, keepdims=True)

        g_lo = jnp.min(btc)
        g_hi = jnp.minimum(jnp.max(btc), _G - 1)

        def gbody(g, carry):
            mask = btc == g
            vals = jnp.where(mask, hb, -jnp.inf)
            mg = jnp.max(vals, axis=0, keepdims=True)
            max_s[pl.ds(g, 1), :] = jnp.maximum(max_s[pl.ds(g, 1), :], mg)
            return carry

        lax.fori_loop(g_lo, g_hi + 1, gbody, 0)

        @pl.when(i == grid - 1)
        def _():
            cnt = jnp.maximum(cnt_s[...], 1.0)
            mean = sum_s[...] / cnt
            z = jnp.dot(mean, f1m_ref[...], preferred_element_type=_F32, precision=_PREC)
            z += jnp.dot(max_s[...], f1x_ref[...], preferred_element_type=_F32, precision=_PREC)
            z = jnp.maximum(z + f1b_ref[...], 0.0)
            o_ref[...] = jnp.dot(
                z, f2w_ref[...], preferred_element_type=_F32, precision=_PREC) + f2b_ref[...]

    return pl.pallas_call(
        body,
        grid=(grid,),
        in_specs=[
            pl.BlockSpec((_BN, _H), lambda i: (i, 0)),
            pl.BlockSpec((1, 1, _BN), lambda i: (i, 0, 0)),
            pl.BlockSpec((1, _BN, 1), lambda i: (i, 0, 0)),
            pl.BlockSpec((_H, _H), lambda i: (0, 0)),
            pl.BlockSpec((_H, _H), lambda i: (0, 0)),
            pl.BlockSpec((1, _H), lambda i: (0, 0)),
            pl.BlockSpec((_H, 1), lambda i: (0, 0)),
            pl.BlockSpec((1, 1), lambda i: (0, 0)),
        ],
        out_specs=pl.BlockSpec((_G, 1), lambda i: (0, 0)),
        out_shape=jax.ShapeDtypeStruct((_G, 1), _F32),
        scratch_shapes=[
            pltpu.VMEM((_G, _H), _F32),
            pltpu.VMEM((_G, _H), _F32),
            pltpu.VMEM((_G, 1), _F32),
        ],
        compiler_params=_tc_params(),
    )(h_pad, brow, bcol, f1m, f1x, f1b.reshape(1, -1), f2w,
      f2b.reshape(1, -1))


def kernel(x, edge_index, edge_attr, batch, params):
    src = edge_index[0]
    dst = edge_index[1]
    pad = _EPAD - _E
    src_p = jnp.concatenate(
        [src, jnp.zeros((pad,), jnp.int32)]).reshape(_NW, _NCHUNK, _CHUNK)
    dst_p = jnp.concatenate(
        [dst, jnp.full((pad,), _N, jnp.int32)]).reshape(_NW, _NCHUNK, _CHUNK)
    ea_p = jnp.concatenate(
        [edge_attr, jnp.zeros((pad, _ED), _F32)], axis=0)
    zblk = jnp.zeros((_CHUNK, _H), _F32)

    h = _tc_dense(x, params['node_proj_w'], params['node_proj_b'])
    for lp in params['layers']:
        xj = _sc_gather(h, src_p)
        m = _tc_edge_mlp(xj, ea_p, lp['m1w'][:_H], lp['m1w'][_H:],
                         lp['m1b'], lp['m2w'], lp['m2b'])
        parts = _sc_scatter_add(m, dst_p, zblk)
        h = _tc_update(h, parts[0, :_N], parts[1, :_N],
                       lp['uw'][:_H], lp['uw'][_H:], lp['ub'],
                       lp['ln_g'], lp['ln_b'])

    npad = _NPAD - _N
    h_pad = jnp.concatenate([h, jnp.zeros((npad, _H), _F32)], axis=0)
    b_pad = jnp.concatenate([batch, jnp.full((npad,), _G, jnp.int32)])
    brow = b_pad.reshape(_NPAD // _BN, 1, _BN)
    bcol = b_pad.reshape(_NPAD // _BN, _BN, 1)
    return _tc_pool_readout(h_pad, brow, bcol,
                            params['f1w'][:_H], params['f1w'][_H:],
                            params['f1b'], params['f2w'], params['f2b'])


# speedup vs baseline: 1.7855x; 1.0055x over previous
import functools

import jax
import jax.numpy as jnp
from jax import lax
from jax.experimental import pallas as pl
from jax.experimental.pallas import tpu as pltpu
from jax.experimental.pallas import tpu_sc as plsc

_N = 10000
_E = 320000
_D = 128
_ED = 16
_H = 128
_G = 64

_NC = 2
_NS = 16
_NW = _NC * _NS
_CHUNK = 128
_EPAD = 327680
_PERW = _EPAD // _NW
_NCHUNK = _PERW // _CHUNK
_NPAD = 10240
_RPT = _NPAD // _NS

_BN = 512
_BE = 1024
_F32 = jnp.float32
_PREC = lax.Precision.DEFAULT


def _mesh():
    return plsc.VectorSubcoreMesh(core_axis_name="c", subcore_axis_name="s")


def _sc_gather(table, idx):
    @functools.partial(
        pl.kernel,
        out_type=jax.ShapeDtypeStruct((_EPAD, _H), _F32),
        mesh=_mesh(),
        scratch_types=[
            pltpu.VMEM((_NCHUNK, _CHUNK), jnp.int32),
            pltpu.VMEM((_CHUNK, _H), _F32),
            pltpu.VMEM((_CHUNK, _H), _F32),
            pltpu.VMEM((_CHUNK, _H), _F32),
            pltpu.VMEM((_CHUNK, _H), _F32),
            pltpu.SemaphoreType.DMA,
            pltpu.SemaphoreType.DMA,
            pltpu.SemaphoreType.DMA,
            pltpu.SemaphoreType.DMA,
        ],
    )
    def k(table_hbm, idx_hbm, out_hbm, idx_v, buf0, buf1, buf2, buf3,
          sem0, sem1, sem2, sem3):
        c = lax.axis_index("c")
        s = lax.axis_index("s")
        wid = s * _NC + c
        base = wid * _PERW
        pltpu.sync_copy(idx_hbm.at[wid], idx_v)
        bufs = (buf0, buf1, buf2, buf3)
        sems = (sem0, sem1, sem2, sem3)

        def start(j, b):
            pltpu.async_copy(table_hbm.at[idx_v.at[j]], bufs[b], sems[b])

        for b in range(4):
            start(b, b)

        def body(jj, carry):
            for b in range(4):
                j = jj * 4 + b
                pltpu.make_async_copy(
                    table_hbm.at[idx_v.at[j]], bufs[b], sems[b]).wait()
                pltpu.sync_copy(
                    bufs[b], out_hbm.at[pl.ds(base + j * _CHUNK, _CHUNK)])

                @pl.when(j + 4 < _NCHUNK)
                def _():
                    start(j + 4, b)
            return carry

        lax.fori_loop(0, _NCHUNK // 4, body, 0)

    return k(table, idx)


def _sc_scatter_add(m, idx, zblk):
    @functools.partial(
        pl.kernel,
        out_type=jax.ShapeDtypeStruct((_NC, _NPAD, _H), _F32),
        mesh=_mesh(),
        scratch_types=[
            pltpu.VMEM((_NCHUNK, _CHUNK), jnp.int32),
            pltpu.VMEM((_CHUNK, _H), _F32),
            pltpu.VMEM((_CHUNK, _H), _F32),
            pltpu.VMEM_SHARED((_NPAD, _H), _F32),
            pltpu.SemaphoreType.DMA,
            pltpu.SemaphoreType.DMA,
        ],
    )
    def k(m_hbm, idx_hbm, z_hbm, out_hbm, idx_v, buf0, buf1, acc, sem0, sem1):
        c = lax.axis_index("c")
        s = lax.axis_index("s")
        wid = s * _NC + c
        base = wid * _PERW
        for r in range(_RPT // _CHUNK):
            pltpu.sync_copy(
                z_hbm, acc.at[pl.ds(s * _RPT + r * _CHUNK, _CHUNK)])
        pltpu.sync_copy(idx_hbm.at[wid], idx_v)
        plsc.subcore_barrier()

        bufs = (buf0, buf1)
        sems = (sem0, sem1)

        def start(j, b):
            pltpu.async_copy(
                m_hbm.at[pl.ds(base + j * _CHUNK, _CHUNK)], bufs[b], sems[b])

        start(0, 0)
        start(1, 1)

        def body(jj, carry):
            for b in range(2):
                j = jj * 2 + b
                pltpu.make_async_copy(
                    m_hbm.at[pl.ds(base + j * _CHUNK, _CHUNK)],
                    bufs[b], sems[b]).wait()
                pltpu.sync_copy(bufs[b], acc.at[idx_v.at[j]], add=True)

                @pl.when(j + 2 < _NCHUNK)
                def _():
                    start(j + 2, b)
            return carry

        lax.fori_loop(0, _NCHUNK // 2, body, 0)
        plsc.subcore_barrier()
        pltpu.sync_copy(
            acc.at[pl.ds(s * _RPT, _RPT)],
            out_hbm.at[c, pl.ds(s * _RPT, _RPT)])

    return k(m, idx, zblk)


def _tc_params():
    return pltpu.CompilerParams(dimension_semantics=("arbitrary",))


def _tc_dense(x, w, b):
    n, din = x.shape
    dout = w.shape[1]

    def body(x_ref, w_ref, b_ref, o_ref):
        o_ref[...] = jnp.dot(
            x_ref[...], w_ref[...], preferred_element_type=_F32, precision=_PREC) + b_ref[...]

    return pl.pallas_call(
        body,
        grid=(pl.cdiv(n, _BN),),
        in_specs=[
            pl.BlockSpec((_BN, din), lambda i: (i, 0)),
            pl.BlockSpec((din, dout), lambda i: (0, 0)),
            pl.BlockSpec((1, dout), lambda i: (0, 0)),
        ],
        out_specs=pl.BlockSpec((_BN, dout), lambda i: (i, 0)),
        out_shape=jax.ShapeDtypeStruct((n, dout), _F32),
        compiler_params=_tc_params(),
    )(x, w, b.reshape(1, -1))


def _tc_edge_mlp(xj, ea, w1x, w1e, b1, w2, b2):
    def body(xj_ref, ea_ref, w1x_ref, w1e_ref, b1_ref, w2_ref, b2_ref, o_ref):
        t = jnp.dot(xj_ref[...], w1x_ref[...], preferred_element_type=_F32, precision=_PREC)
        t += jnp.dot(ea_ref[...], w1e_ref[...], preferred_element_type=_F32, precision=_PREC)
        t = jnp.maximum(t + b1_ref[...], 0.0)
        o_ref[...] = jnp.dot(
            t, w2_ref[...], preferred_element_type=_F32, precision=_PREC) + b2_ref[...]

    return pl.pallas_call(
        body,
        grid=(_EPAD // _BE,),
        in_specs=[
            pl.BlockSpec((_BE, _H), lambda i: (i, 0)),
            pl.BlockSpec((_BE, _ED), lambda i: (i, 0)),
            pl.BlockSpec((_H, _H), lambda i: (0, 0)),
            pl.BlockSpec((_ED, _H), lambda i: (0, 0)),
            pl.BlockSpec((1, _H), lambda i: (0, 0)),
            pl.BlockSpec((_H, _H), lambda i: (0, 0)),
            pl.BlockSpec((1, _H), lambda i: (0, 0)),
        ],
        out_specs=pl.BlockSpec((_BE, _H), lambda i: (i, 0)),
        out_shape=jax.ShapeDtypeStruct((_EPAD, _H), _F32),
        compiler_params=_tc_params(),
    )(xj, ea, w1x, w1e, b1.reshape(1, -1), w2, b2.reshape(1, -1))


def _tc_update(h, a0, a1, uwh, uwa, ub, lng, lnb):
    def body(h_ref, a0_ref, a1_ref, uwh_ref, uwa_ref, ub_ref, g_ref, b_ref,
             o_ref):
        hb = h_ref[...]
        ag = a0_ref[...] + a1_ref[...]
        u = jnp.dot(hb, uwh_ref[...], preferred_element_type=_F32, precision=_PREC)
        u += jnp.dot(ag, uwa_ref[...], preferred_element_type=_F32, precision=_PREC)
        u = jnp.maximum(u + ub_ref[...], 0.0)
        mu = jnp.mean(u, axis=1, keepdims=True)
        var = jnp.mean((u - mu) ** 2, axis=1, keepdims=True)
        hn = (u - mu) / jnp.sqrt(var + 1e-5) * g_ref[...] + b_ref[...]
        o_ref[...] = jnp.maximum(hn, 0.0) + hb

    return pl.pallas_call(
        body,
        grid=(pl.cdiv(_N, _BN),),
        in_specs=[
            pl.BlockSpec((_BN, _H), lambda i: (i, 0)),
            pl.BlockSpec((_BN, _H), lambda i: (i, 0)),
            pl.BlockSpec((_BN, _H), lambda i: (i, 0)),
            pl.BlockSpec((_H, _H), lambda i: (0, 0)),
            pl.BlockSpec((_H, _H), lambda i: (0, 0)),
            pl.BlockSpec((1, _H), lambda i: (0, 0)),
            pl.BlockSpec((1, _H), lambda i: (0, 0)),
            pl.BlockSpec((1, _H), lambda i: (0, 0)),
        ],
        out_specs=pl.BlockSpec((_BN, _H), lambda i: (i, 0)),
        out_shape=jax.ShapeDtypeStruct((_N, _H), _F32),
        compiler_params=_tc_params(),
    )(h, a0, a1, uwh, uwa, ub.reshape(1, -1), lng.reshape(1, -1),
      lnb.reshape(1, -1))


def _tc_pool_readout(h_pad, brow, bcol, f1m, f1x, f1b, f2w, f2b):
    grid = _NPAD // _BN

    def body(h_ref, br_ref, bc_ref, f1m_ref, f1x_ref, f1b_ref, f2w_ref,
             f2b_ref, o_ref, sum_s, max_s, cnt_s):
        i = pl.program_id(0)

        @pl.when(i == 0)
        def _():
            sum_s[...] = jnp.zeros_like(sum_s)
            max_s[...] = jnp.full_like(max_s, -jnp.inf)
            cnt_s[...] = jnp.zeros_like(cnt_s)

        hb = h_ref[...]
        bt = br_ref[0]
        btc = bc_ref[0]
        gi = lax.broadcasted_iota(jnp.int32, (_G, 1), 0)
        onehot = (gi == bt).astype(_F32)
        sum_s[...] += jnp.dot(onehot, hb, preferred_element_type=_F32,
                              precision=lax.Precision.HIGHEST)
        cnt_s[...] += jnp.sum(onehot, axis=1, keepdims=True)

        g_lo = jnp.min(btc)
        g_hi = jnp.minimum(jnp.max(btc), _G - 1)

        def gbody(g, carry):
            mask = btc == g
            vals = jnp.where(mask, hb, -jnp.inf)
            mg = jnp.max(vals, axis=0, keepdims=True)
            max_s[pl.ds(g, 1), :] = jnp.maximum(max_s[pl.ds(g, 1), :], mg)
            return carry

        lax.fori_loop(g_lo, g_hi + 1, gbody, 0)

        @pl.when(i == grid - 1)
        def _():
            cnt = jnp.maximum(cnt_s[...], 1.0)
            mean = sum_s[...] / cnt
            z = jnp.dot(mean, f1m_ref[...], preferred_element_type=_F32, precision=_PREC)
            z += jnp.dot(max_s[...], f1x_ref[...], preferred_element_type=_F32, precision=_PREC)
            z = jnp.maximum(z + f1b_ref[...], 0.0)
            o_ref[...] = jnp.dot(
                z, f2w_ref[...], preferred_element_type=_F32, precision=_PREC) + f2b_ref[...]

    return pl.pallas_call(
        body,
        grid=(grid,),
        in_specs=[
            pl.BlockSpec((_BN, _H), lambda i: (i, 0)),
            pl.BlockSpec((1, 1, _BN), lambda i: (i, 0, 0)),
            pl.BlockSpec((1, _BN, 1), lambda i: (i, 0, 0)),
            pl.BlockSpec((_H, _H), lambda i: (0, 0)),
            pl.BlockSpec((_H, _H), lambda i: (0, 0)),
            pl.BlockSpec((1, _H), lambda i: (0, 0)),
            pl.BlockSpec((_H, 1), lambda i: (0, 0)),
            pl.BlockSpec((1, 1), lambda i: (0, 0)),
        ],
        out_specs=pl.BlockSpec((_G, 1), lambda i: (0, 0)),
        out_shape=jax.ShapeDtypeStruct((_G, 1), _F32),
        scratch_shapes=[
            pltpu.VMEM((_G, _H), _F32),
            pltpu.VMEM((_G, _H), _F32),
            pltpu.VMEM((_G, 1), _F32),
        ],
        compiler_params=_tc_params(),
    )(h_pad, brow, bcol, f1m, f1x, f1b.reshape(1, -1), f2w,
      f2b.reshape(1, -1))


def kernel(x, edge_index, edge_attr, batch, params):
    src = edge_index[0]
    dst = edge_index[1]
    pad = _EPAD - _E
    src_p = jnp.concatenate(
        [src, jnp.zeros((pad,), jnp.int32)]).reshape(_NW, _NCHUNK, _CHUNK)
    dst_p = jnp.concatenate(
        [dst, jnp.full((pad,), _N, jnp.int32)]).reshape(_NW, _NCHUNK, _CHUNK)
    ea_p = jnp.concatenate(
        [edge_attr, jnp.zeros((pad, _ED), _F32)], axis=0)
    zblk = jnp.zeros((_CHUNK, _H), _F32)

    h = _tc_dense(x, params['node_proj_w'], params['node_proj_b'])
    for lp in params['layers']:
        xj = _sc_gather(h, src_p)
        m = _tc_edge_mlp(xj, ea_p, lp['m1w'][:_H], lp['m1w'][_H:],
                         lp['m1b'], lp['m2w'], lp['m2b'])
        parts = _sc_scatter_add(m, dst_p, zblk)
        h = _tc_update(h, parts[0, :_N], parts[1, :_N],
                       lp['uw'][:_H], lp['uw'][_H:], lp['ub'],
                       lp['ln_g'], lp['ln_b'])

    npad = _NPAD - _N
    h_pad = jnp.concatenate([h, jnp.zeros((npad, _H), _F32)], axis=0)
    b_pad = jnp.concatenate([batch, jnp.full((npad,), _G, jnp.int32)])
    brow = b_pad.reshape(_NPAD // _BN, 1, _BN)
    bcol = b_pad.reshape(_NPAD // _BN, _BN, 1)
    return _tc_pool_readout(h_pad, brow, bcol,
                            params['f1w'][:_H], params['f1w'][_H:],
                            params['f1b'], params['f2w'], params['f2b'])
